# Initial kernel scaffold; baseline (speedup 1.0000x reference)
import functools

import jax
import jax.numpy as jnp
from jax.experimental import pallas as pl
from jax.experimental.pallas import tpu as pltpu
from jax.experimental.pallas import tpu_sc as plsc

NC = 2
NS = 16
NW = NC * NS
LANES = 16
CHUNK = 128


def _degree_kernel(n_nodes, n_edges):
  e_tile = n_edges // NW
  n_full = e_tile // CHUNK
  tail = e_tile - n_full * CHUNK
  rows_tile = n_nodes // NS

  mesh = plsc.VectorSubcoreMesh(core_axis_name="c", subcore_axis_name="s",
                                num_cores=NC, num_subcores=NS)

  @functools.partial(
      pl.kernel,
      out_type=jax.ShapeDtypeStruct((NC, n_nodes, LANES), jnp.float32),
      mesh=mesh,
      scratch_types=[
          pltpu.VMEM((CHUNK,), jnp.int32),
          pltpu.VMEM((CHUNK, LANES), jnp.float32),
          pltpu.VMEM((n_nodes // NS, LANES), jnp.float32),
          pltpu.VMEM_SHARED((n_nodes, LANES), jnp.float32),
      ],
  )
  def k(dst_hbm, degp_hbm, dst_v, ones_v, zero_v, deg_sh):
    c = jax.lax.axis_index("c")
    s = jax.lax.axis_index("s")
    wid = c * NS + s
    ebase = wid * e_tile
    rbase = s * rows_tile

    @pl.loop(0, CHUNK)
    def _(i):
      ones_v[i, :] = jnp.ones((LANES,), jnp.float32)

    @pl.loop(0, rows_tile)
    def _(i):
      zero_v[i, :] = jnp.zeros((LANES,), jnp.float32)

    pltpu.sync_copy(zero_v, deg_sh.at[pl.ds(rbase, rows_tile)])
    plsc.subcore_barrier()

    @pl.loop(0, n_full)
    def _(j):
      pltpu.sync_copy(dst_hbm.at[pl.ds(ebase + j * CHUNK, CHUNK)], dst_v)
      pltpu.sync_copy(ones_v, deg_sh.at[dst_v], add=True)

    if tail:
      pltpu.sync_copy(dst_hbm.at[pl.ds(ebase + n_full * CHUNK, tail)],
                      dst_v.at[pl.ds(0, tail)])
      pltpu.sync_copy(ones_v.at[pl.ds(0, tail)],
                      deg_sh.at[dst_v.at[pl.ds(0, tail)]], add=True)

    plsc.subcore_barrier()
    pltpu.sync_copy(deg_sh.at[pl.ds(rbase, rows_tile)],
                    degp_hbm.at[c].at[pl.ds(rbase, rows_tile)])

  return k


def _scatter_kernel(n_nodes, n_edges, d):
  e_tile = n_edges // NW
  n_full = e_tile // CHUNK
  tail = e_tile - n_full * CHUNK
  rows_tile = n_nodes // NS
  zrows = 125

  mesh = plsc.VectorSubcoreMesh(core_axis_name="c", subcore_axis_name="s",
                                num_cores=NC, num_subcores=NS)

  @functools.partial(
      pl.kernel,
      out_type=jax.ShapeDtypeStruct((NC, n_nodes, d), jnp.float32),
      mesh=mesh,
      scratch_types=[
          pltpu.VMEM((CHUNK,), jnp.int32),
          pltpu.VMEM((CHUNK,), jnp.int32),
          pltpu.VMEM((CHUNK, d), jnp.float32),
          pltpu.VMEM_SHARED((n_nodes, d), jnp.float32),
      ],
  )
  def k(u_hbm, src_hbm, dst_hbm, vp_hbm, src_v, dst_v, rows_v, v_sh):
    c = jax.lax.axis_index("c")
    s = jax.lax.axis_index("s")
    wid = c * NS + s
    ebase = wid * e_tile
    rbase = s * rows_tile

    @pl.loop(0, zrows)
    def _(i):
      @pl.loop(0, d, step=LANES)
      def _(jj):
        rows_v[i, pl.ds(jj, LANES)] = jnp.zeros((LANES,), jnp.float32)

    @pl.loop(0, rows_tile // zrows)
    def _(t):
      pltpu.sync_copy(rows_v.at[pl.ds(0, zrows)],
                      v_sh.at[pl.ds(rbase + t * zrows, zrows)])
    rem = rows_tile % zrows
    if rem:
      pltpu.sync_copy(rows_v.at[pl.ds(0, rem)],
                      v_sh.at[pl.ds(rbase + (rows_tile // zrows) * zrows, rem)])

    plsc.subcore_barrier()

    @pl.loop(0, n_full)
    def _(j):
      base = ebase + j * CHUNK
      pltpu.sync_copy(src_hbm.at[pl.ds(base, CHUNK)], src_v)
      pltpu.sync_copy(dst_hbm.at[pl.ds(base, CHUNK)], dst_v)
      pltpu.sync_copy(u_hbm.at[src_v], rows_v)
      pltpu.sync_copy(rows_v, v_sh.at[dst_v], add=True)

    if tail:
      base = ebase + n_full * CHUNK
      pltpu.sync_copy(src_hbm.at[pl.ds(base, tail)], src_v.at[pl.ds(0, tail)])
      pltpu.sync_copy(dst_hbm.at[pl.ds(base, tail)], dst_v.at[pl.ds(0, tail)])
      pltpu.sync_copy(u_hbm.at[src_v.at[pl.ds(0, tail)]],
                      rows_v.at[pl.ds(0, tail)])
      pltpu.sync_copy(rows_v.at[pl.ds(0, tail)],
                      v_sh.at[dst_v.at[pl.ds(0, tail)]], add=True)

    plsc.subcore_barrier()
    pltpu.sync_copy(v_sh.at[pl.ds(rbase, rows_tile)],
                    vp_hbm.at[c].at[pl.ds(rbase, rows_tile)])

  return k


def _scale_body(degp_ref, x_ref, u_ref):
  deg = degp_ref[0, :, 0:1] + degp_ref[1, :, 0:1] + 1.0
  u_ref[...] = x_ref[...] * jax.lax.rsqrt(deg)


def _combine_body(vp_ref, u_ref, degp_ref, w_ref, b_ref, o_ref):
  deg = degp_ref[0, :, 0:1] + degp_ref[1, :, 0:1] + 1.0
  z = (vp_ref[0] + vp_ref[1] + u_ref[...]) * jax.lax.rsqrt(deg)
  o_ref[...] = (
      jnp.dot(z, w_ref[...], preferred_element_type=jnp.float32) + b_ref[...]
  )


@jax.jit
def kernel(x, edge_index, W, b):
  n, d = x.shape
  e = edge_index.shape[1]
  src = edge_index[0].astype(jnp.int32)
  dst = edge_index[1].astype(jnp.int32)

  degp = _degree_kernel(n, e)(dst)
  u = pl.pallas_call(
      _scale_body,
      out_shape=jax.ShapeDtypeStruct((n, d), jnp.float32),
  )(degp, x)
  vp = _scatter_kernel(n, e, d)(u, src, dst)
  out = pl.pallas_call(
      _combine_body,
      out_shape=jax.ShapeDtypeStruct((n, d), jnp.float32),
  )(vp, u, degp, W, b.reshape(1, d))
  return out


# speedup vs baseline: 21.9176x; 21.9176x over previous
import functools

import jax
import jax.numpy as jnp
from jax.experimental import pallas as pl
from jax.experimental.pallas import tpu as pltpu
from jax.experimental.pallas import tpu_sc as plsc

NC = 2
NS = 16
NW = NC * NS
LANES = 16
CHUNK = 128


def _padded_rows(n_nodes):
  per = -(-n_nodes // (NS * 8)) * 8
  return per, per * NS


def _zero_shared_rows(stage_v, sh, rbase, rows, width):
  @pl.loop(0, CHUNK)
  def _(i):
    @pl.loop(0, width, step=LANES)
    def _(jj):
      stage_v[i, pl.ds(jj, LANES)] = jnp.zeros((LANES,), jnp.float32)

  @pl.loop(0, rows // CHUNK)
  def _(t):
    pltpu.sync_copy(stage_v, sh.at[pl.ds(rbase + t * CHUNK, CHUNK)])
  rem = rows % CHUNK
  if rem:
    pltpu.sync_copy(stage_v.at[pl.ds(0, rem)],
                    sh.at[pl.ds(rbase + (rows // CHUNK) * CHUNK, rem)])


def _degree_kernel(n_nodes, n_edges):
  e_tile = n_edges // NW
  n_full = e_tile // CHUNK
  tail = e_tile - n_full * CHUNK
  rows_tile, np_rows = _padded_rows(n_nodes)

  mesh = plsc.VectorSubcoreMesh(core_axis_name="c", subcore_axis_name="s",
                                num_cores=NC, num_subcores=NS)

  scratch = [
      pltpu.VMEM((CHUNK,), jnp.int32),
      pltpu.VMEM((CHUNK, LANES), jnp.float32),
      pltpu.VMEM_SHARED((np_rows, LANES), jnp.float32),
  ]
  if tail:
    scratch.append(pltpu.VMEM((tail,), jnp.int32))

  @functools.partial(
      pl.kernel,
      out_type=jax.ShapeDtypeStruct((NC, np_rows, LANES), jnp.float32),
      mesh=mesh,
      scratch_types=scratch,
  )
  def k(dst_hbm, degp_hbm, dst_v, ones_v, deg_sh, *maybe_tail):
    c = jax.lax.axis_index("c")
    s = jax.lax.axis_index("s")
    wid = c * NS + s
    ebase = wid * e_tile
    rbase = s * rows_tile

    _zero_shared_rows(ones_v, deg_sh, rbase, rows_tile, LANES)

    @pl.loop(0, CHUNK)
    def _(i):
      ones_v[i, :] = jnp.ones((LANES,), jnp.float32)

    plsc.subcore_barrier()

    @pl.loop(0, n_full)
    def _(j):
      pltpu.sync_copy(dst_hbm.at[pl.ds(ebase + j * CHUNK, CHUNK)], dst_v)
      pltpu.sync_copy(ones_v, deg_sh.at[dst_v], add=True)

    if tail:
      dst_t = maybe_tail[0]
      pltpu.sync_copy(dst_hbm.at[pl.ds(ebase + n_full * CHUNK, tail)], dst_t)
      pltpu.sync_copy(ones_v.at[pl.ds(0, tail)], deg_sh.at[dst_t], add=True)

    plsc.subcore_barrier()
    pltpu.sync_copy(deg_sh.at[pl.ds(rbase, rows_tile)],
                    degp_hbm.at[c].at[pl.ds(rbase, rows_tile)])

  return k


def _scatter_kernel(n_nodes, n_edges, d):
  e_tile = n_edges // NW
  n_full = e_tile // CHUNK
  tail = e_tile - n_full * CHUNK
  rows_tile, np_rows = _padded_rows(n_nodes)

  mesh = plsc.VectorSubcoreMesh(core_axis_name="c", subcore_axis_name="s",
                                num_cores=NC, num_subcores=NS)

  scratch = [
      pltpu.VMEM((CHUNK,), jnp.int32),
      pltpu.VMEM((CHUNK,), jnp.int32),
      pltpu.VMEM((CHUNK, d), jnp.float32),
      pltpu.VMEM_SHARED((np_rows, d), jnp.float32),
  ]
  if tail:
    scratch += [pltpu.VMEM((tail,), jnp.int32), pltpu.VMEM((tail,), jnp.int32)]

  @functools.partial(
      pl.kernel,
      out_type=jax.ShapeDtypeStruct((NC, np_rows, d), jnp.float32),
      mesh=mesh,
      scratch_types=scratch,
  )
  def k(u_hbm, src_hbm, dst_hbm, vp_hbm, src_v, dst_v, rows_v, v_sh,
        *maybe_tail):
    c = jax.lax.axis_index("c")
    s = jax.lax.axis_index("s")
    wid = c * NS + s
    ebase = wid * e_tile
    rbase = s * rows_tile

    _zero_shared_rows(rows_v, v_sh, rbase, rows_tile, d)
    plsc.subcore_barrier()

    @pl.loop(0, n_full)
    def _(j):
      base = ebase + j * CHUNK
      pltpu.sync_copy(src_hbm.at[pl.ds(base, CHUNK)], src_v)
      pltpu.sync_copy(dst_hbm.at[pl.ds(base, CHUNK)], dst_v)
      pltpu.sync_copy(u_hbm.at[src_v], rows_v)
      pltpu.sync_copy(rows_v, v_sh.at[dst_v], add=True)

    if tail:
      src_t, dst_t = maybe_tail
      base = ebase + n_full * CHUNK
      pltpu.sync_copy(src_hbm.at[pl.ds(base, tail)], src_t)
      pltpu.sync_copy(dst_hbm.at[pl.ds(base, tail)], dst_t)
      pltpu.sync_copy(u_hbm.at[src_t], rows_v.at[pl.ds(0, tail)])
      pltpu.sync_copy(rows_v.at[pl.ds(0, tail)], v_sh.at[dst_t], add=True)

    plsc.subcore_barrier()
    pltpu.sync_copy(v_sh.at[pl.ds(rbase, rows_tile)],
                    vp_hbm.at[c].at[pl.ds(rbase, rows_tile)])

  return k


def _scale_body(n, degp_ref, x_ref, u_ref):
  deg = degp_ref[0, 0:n, 0:1] + degp_ref[1, 0:n, 0:1] + 1.0
  u_ref[...] = x_ref[...] * jax.lax.rsqrt(deg)


def _combine_body(n, vp_ref, u_ref, degp_ref, w_ref, b_ref, o_ref):
  deg = degp_ref[0, 0:n, 0:1] + degp_ref[1, 0:n, 0:1] + 1.0
  z = (vp_ref[0, 0:n, :] + vp_ref[1, 0:n, :] + u_ref[...]) * jax.lax.rsqrt(deg)
  o_ref[...] = (
      jnp.dot(z, w_ref[...], preferred_element_type=jnp.float32) + b_ref[...]
  )


@jax.jit
def kernel(x, edge_index, W, b):
  n, d = x.shape
  e = edge_index.shape[1]
  src = edge_index[0].astype(jnp.int32)
  dst = edge_index[1].astype(jnp.int32)

  degp = _degree_kernel(n, e)(dst)
  u = pl.pallas_call(
      functools.partial(_scale_body, n),
      out_shape=jax.ShapeDtypeStruct((n, d), jnp.float32),
  )(degp, x)
  vp = _scatter_kernel(n, e, d)(u, src, dst)
  out = pl.pallas_call(
      functools.partial(_combine_body, n),
      out_shape=jax.ShapeDtypeStruct((n, d), jnp.float32),
  )(vp, u, degp, W, b.reshape(1, d))
  return out
